# Initial kernel scaffold; baseline (speedup 1.0000x reference)
import jax
import jax.numpy as jnp
from jax.experimental import pallas as pl


def kernel(x, edge_index, W1, b1, W2, b2, Wc, bc):
    raise NotImplementedError("write your pallas kernel here")


# speedup vs baseline: 84.1535x; 84.1535x over previous
import functools

import jax
import jax.numpy as jnp
from jax import lax
from jax.experimental import pallas as pl
from jax.experimental.pallas import tpu as pltpu
from jax.experimental.pallas import tpu_sc as plsc

N = 10000
E = 320000
D = 128
H = 64
C = 2

NUM_CORES = 2
NUM_SUBCORES = 16
NW = NUM_CORES * NUM_SUBCORES
E_PER_W = E // NW
LANES = 16


def _dense_body(x_ref, w1_ref, b1_ref, w2_ref, b2_ref, wc_ref, bc_ref,
                att_ref, z_ref, info_ref):
    x = x_ref[...]
    h = jnp.maximum(jnp.dot(x, w1_ref[...], preferred_element_type=jnp.float32)
                    + b1_ref[...][None, :], 0.0)
    logit = jnp.dot(h, w2_ref[...], preferred_element_type=jnp.float32) + b2_ref[...][None, :]
    att = jax.nn.sigmoid(logit)
    att_ref[...] = att
    z_ref[...] = jnp.dot(x * att, wc_ref[...], preferred_element_type=jnp.float32)
    r = 0.7
    t = att * jnp.log(att / r + 1e-06) + (1.0 - att) * jnp.log((1.0 - att) / (1.0 - r + 1e-06) + 1e-06)
    info_ref[...] = jnp.reshape(jnp.sum(t) / float(N), (1, 1))


def _dense_head(x, W1, b1, W2, b2, Wc, bc):
    return pl.pallas_call(
        _dense_body,
        out_shape=(
            jax.ShapeDtypeStruct((N, 1), jnp.float32),
            jax.ShapeDtypeStruct((N, C), jnp.float32),
            jax.ShapeDtypeStruct((1, 1), jnp.float32),
        ),
    )(x, W1, b1, W2, b2, Wc, bc)


def _sc_body(ei_hbm, att_hbm, z0_hbm, z1_hbm,
             ea_hbm, p0_hbm, p1_hbm,
             att_v, z0_v, z1_v, src_v, dst_v, ea_v, acc0_v, acc1_v):
    wid = lax.axis_index("s") * NUM_CORES + lax.axis_index("c")
    base = wid * E_PER_W
    pltpu.sync_copy(att_hbm, att_v)
    pltpu.sync_copy(z0_hbm, z0_v)
    pltpu.sync_copy(z1_hbm, z1_v)
    pltpu.sync_copy(ei_hbm.at[pl.ds(base, E_PER_W)], src_v)
    pltpu.sync_copy(ei_hbm.at[pl.ds(E + base, E_PER_W)], dst_v)

    def zero_body(i, carry):
        zv = jnp.zeros((LANES,), jnp.float32)
        acc0_v[pl.ds(i * LANES, LANES)] = zv
        acc1_v[pl.ds(i * LANES, LANES)] = zv
        return carry

    lax.fori_loop(0, N // LANES, zero_body, 0)

    def edge_body(i, carry):
        s = src_v[pl.ds(i * LANES, LANES)]
        t = dst_v[pl.ds(i * LANES, LANES)]
        a_s = plsc.load_gather(att_v, [s])
        a_t = plsc.load_gather(att_v, [t])
        ea_v[pl.ds(i * LANES, LANES)] = a_s * a_t
        zs0 = plsc.load_gather(z0_v, [s])
        zs1 = plsc.load_gather(z1_v, [s])
        plsc.addupdate_scatter(acc0_v, [t], zs0)
        plsc.addupdate_scatter(acc1_v, [t], zs1)
        return carry

    lax.fori_loop(0, E_PER_W // LANES, edge_body, 0)

    pltpu.sync_copy(ea_v, ea_hbm.at[pl.ds(base, E_PER_W)])
    pltpu.sync_copy(acc0_v, p0_hbm.at[wid])
    pltpu.sync_copy(acc1_v, p1_hbm.at[wid])


_sc_edges = functools.partial(
    pl.kernel,
    out_type=(
        jax.ShapeDtypeStruct((E,), jnp.float32),
        jax.ShapeDtypeStruct((NW, N), jnp.float32),
        jax.ShapeDtypeStruct((NW, N), jnp.float32),
    ),
    mesh=plsc.VectorSubcoreMesh(core_axis_name="c", subcore_axis_name="s"),
    compiler_params=pltpu.CompilerParams(needs_layout_passes=False),
    scratch_types=[
        pltpu.VMEM((N,), jnp.float32),
        pltpu.VMEM((N,), jnp.float32),
        pltpu.VMEM((N,), jnp.float32),
        pltpu.VMEM((E_PER_W,), jnp.int32),
        pltpu.VMEM((E_PER_W,), jnp.int32),
        pltpu.VMEM((E_PER_W,), jnp.float32),
        pltpu.VMEM((N,), jnp.float32),
        pltpu.VMEM((N,), jnp.float32),
    ],
)(_sc_body)


def _combine_body(p0_ref, p1_ref, att_ref, bc_ref, c0_ref, c1_ref):
    s0 = jnp.sum(p0_ref[...], axis=0)
    s1 = jnp.sum(p1_ref[...], axis=0)
    att = att_ref[...]
    c0_ref[...] = att * s0 + bc_ref[0]
    c1_ref[...] = att * s1 + bc_ref[1]


def _combine(p0, p1, att1, bc):
    return pl.pallas_call(
        _combine_body,
        in_specs=[
            pl.BlockSpec(memory_space=pltpu.VMEM),
            pl.BlockSpec(memory_space=pltpu.VMEM),
            pl.BlockSpec(memory_space=pltpu.VMEM),
            pl.BlockSpec(memory_space=pltpu.SMEM),
        ],
        out_shape=(
            jax.ShapeDtypeStruct((N,), jnp.float32),
            jax.ShapeDtypeStruct((N,), jnp.float32),
        ),
    )(p0, p1, att1, bc)


def kernel(x, edge_index, W1, b1, W2, b2, Wc, bc):
    att2, z, info = _dense_head(x, W1, b1, W2, b2, Wc, bc)
    att1 = att2[:, 0]
    z0 = z[:, 0]
    z1 = z[:, 1]
    edge_att, p0, p1 = _sc_edges(edge_index.reshape(-1), att1, z0, z1)
    c0, c1 = _combine(p0, p1, att1, bc)
    clf_logits = jnp.stack([c0, c1], axis=-1)
    return clf_logits, edge_att, info[0, 0]


# speedup vs baseline: 116.8063x; 1.3880x over previous
import functools

import jax
import jax.numpy as jnp
from jax import lax
from jax.experimental import pallas as pl
from jax.experimental.pallas import tpu as pltpu
from jax.experimental.pallas import tpu_sc as plsc

N = 10000
E = 320000
D = 128
H = 64
C = 2

NUM_CORES = 2
NUM_SUBCORES = 16
NW = NUM_CORES * NUM_SUBCORES
LANES = 16

EDGE_TILE = 128
NTILES = E // EDGE_TILE
BASE_TILES = NTILES // NW
EXTRA = NTILES - BASE_TILES * NW
MAX_EDGES = (BASE_TILES + 1) * EDGE_TILE
BASE_EDGES = BASE_TILES * EDGE_TILE


def _dense_body(x_ref, w1_ref, b1_ref, w2_ref, b2_ref, wc_ref,
                att_ref, z0_ref, z1_ref, info_ref):
    x = x_ref[...]
    hT = jnp.maximum(
        lax.dot_general(w1_ref[...], x, (((0,), (1,)), ((), ())),
                        preferred_element_type=jnp.float32)
        + b1_ref[...][:, None], 0.0)
    logit = lax.dot_general(w2_ref[...], hT, (((0,), (0,)), ((), ())),
                            preferred_element_type=jnp.float32) + b2_ref[0]
    att = jax.nn.sigmoid(logit)
    xwT = lax.dot_general(wc_ref[...], x, (((0,), (1,)), ((), ())),
                          preferred_element_type=jnp.float32)
    att_ref[...] = att.reshape(N)
    z0_ref[...] = (att * xwT[0:1, :]).reshape(N)
    z1_ref[...] = (att * xwT[1:2, :]).reshape(N)
    r = 0.7
    t = att * jnp.log(att / r + 1e-06) + (1.0 - att) * jnp.log((1.0 - att) / (1.0 - r + 1e-06) + 1e-06)
    info_ref[...] = jnp.reshape(jnp.sum(t) / float(N), (1, 1))


def _dense_head(x, W1, b1, W2, b2, Wc):
    return pl.pallas_call(
        _dense_body,
        out_shape=(
            jax.ShapeDtypeStruct((N,), jnp.float32),
            jax.ShapeDtypeStruct((N,), jnp.float32),
            jax.ShapeDtypeStruct((N,), jnp.float32),
            jax.ShapeDtypeStruct((1, 1), jnp.float32),
        ),
    )(x, W1, b1, W2, b2, Wc)


def _sc_body(ei_hbm, att_hbm, z0_hbm, z1_hbm,
             ea_hbm, p0_hbm, p1_hbm,
             ei_v, att_v, z0_v, z1_v, ea_v, acc0_v, acc1_v, sem):
    wid = lax.axis_index("s") * NUM_CORES + lax.axis_index("c")
    ntiles = BASE_TILES + jnp.where(wid < EXTRA, 1, 0)
    base_tile = BASE_TILES * wid + jnp.minimum(wid, EXTRA)
    base_ed = base_tile * EDGE_TILE
    base_eff = jnp.minimum(base_ed, E - MAX_EDGES)
    off = base_ed - base_eff

    cps = [
        pltpu.async_copy(ei_hbm.at[:, pl.ds(base_eff, MAX_EDGES)], ei_v, sem),
        pltpu.async_copy(att_hbm, att_v, sem),
        pltpu.async_copy(z0_hbm, z0_v, sem),
        pltpu.async_copy(z1_hbm, z1_v, sem),
    ]

    def zero_body(i, carry):
        zv = jnp.zeros((LANES,), jnp.float32)
        for k in range(5):
            acc0_v[pl.ds((i * 5 + k) * LANES, LANES)] = zv
            acc1_v[pl.ds((i * 5 + k) * LANES, LANES)] = zv
        return carry

    lax.fori_loop(0, N // (LANES * 5), zero_body, 0)
    for cp in cps:
        cp.wait()

    def edge_body(i, carry):
        for k in range(EDGE_TILE // LANES):
            start = off + i * EDGE_TILE + k * LANES
            s = ei_v[0, pl.ds(start, LANES)]
            t = ei_v[1, pl.ds(start, LANES)]
            a_s = plsc.load_gather(att_v, [s])
            a_t = plsc.load_gather(att_v, [t])
            ea_v[pl.ds(start, LANES)] = a_s * a_t
            zs0 = plsc.load_gather(z0_v, [s])
            zs1 = plsc.load_gather(z1_v, [s])
            plsc.addupdate_scatter(acc0_v, [t], zs0)
            plsc.addupdate_scatter(acc1_v, [t], zs1)
        return carry

    lax.fori_loop(0, ntiles, edge_body, 0)

    pltpu.sync_copy(ea_v.at[pl.ds(off, BASE_EDGES)],
                    ea_hbm.at[pl.ds(base_ed, BASE_EDGES)])

    @pl.when(ntiles == BASE_TILES + 1)
    def _():
        pltpu.sync_copy(ea_v.at[pl.ds(off + BASE_EDGES, EDGE_TILE)],
                        ea_hbm.at[pl.ds(base_ed + BASE_EDGES, EDGE_TILE)])

    pltpu.sync_copy(acc0_v, p0_hbm.at[wid])
    pltpu.sync_copy(acc1_v, p1_hbm.at[wid])


_sc_edges = functools.partial(
    pl.kernel,
    out_type=(
        jax.ShapeDtypeStruct((E,), jnp.float32),
        jax.ShapeDtypeStruct((NW, N), jnp.float32),
        jax.ShapeDtypeStruct((NW, N), jnp.float32),
    ),
    mesh=plsc.VectorSubcoreMesh(core_axis_name="c", subcore_axis_name="s"),
    compiler_params=pltpu.CompilerParams(needs_layout_passes=False),
    scratch_types=[
        pltpu.VMEM((2, MAX_EDGES), jnp.int32),
        pltpu.VMEM((N,), jnp.float32),
        pltpu.VMEM((N,), jnp.float32),
        pltpu.VMEM((N,), jnp.float32),
        pltpu.VMEM((MAX_EDGES,), jnp.float32),
        pltpu.VMEM((N,), jnp.float32),
        pltpu.VMEM((N,), jnp.float32),
        pltpu.SemaphoreType.DMA,
    ],
)(_sc_body)


def _combine_body(p0_ref, p1_ref, att_ref, bc_ref, clf_ref):
    att = att_ref[...][None, :]
    K8 = 2 * NW + 8
    pa = jnp.concatenate(
        [p0_ref[...] * att, p1_ref[...] * att,
         jnp.ones((1, N), jnp.float32),
         jnp.zeros((7, N), jnp.float32)], axis=0)
    rows = lax.broadcasted_iota(jnp.int32, (K8, C), 0)
    cols = lax.broadcasted_iota(jnp.int32, (K8, C), 1)
    bc_row = jnp.where(cols == 0, bc_ref[0], bc_ref[1])
    sel = jnp.where(rows == 2 * NW, bc_row,
                    jnp.where(rows > 2 * NW, 0.0,
                              jnp.where((rows < NW) == (cols == 0), 1.0, 0.0)))
    clf_ref[...] = lax.dot_general(pa, sel, (((0,), (0,)), ((), ())),
                                   preferred_element_type=jnp.float32)


def _combine(p0, p1, att1, bc):
    return pl.pallas_call(
        _combine_body,
        in_specs=[
            pl.BlockSpec(memory_space=pltpu.VMEM),
            pl.BlockSpec(memory_space=pltpu.VMEM),
            pl.BlockSpec(memory_space=pltpu.VMEM),
            pl.BlockSpec(memory_space=pltpu.SMEM),
        ],
        out_shape=jax.ShapeDtypeStruct((N, C), jnp.float32),
    )(p0, p1, att1, bc)


def kernel(x, edge_index, W1, b1, W2, b2, Wc, bc):
    att1, z0, z1, info = _dense_head(x, W1, b1, W2, b2, Wc)
    edge_att, p0, p1 = _sc_edges(edge_index, att1, z0, z1)
    clf_logits = _combine(p0, p1, att1, bc)
    return clf_logits, edge_att, info[0, 0]


# speedup vs baseline: 144.1074x; 1.2337x over previous
import functools

import jax
import jax.numpy as jnp
from jax import lax
from jax.experimental import pallas as pl
from jax.experimental.pallas import tpu as pltpu
from jax.experimental.pallas import tpu_sc as plsc

N = 10000
E = 320000
D = 128
H = 64
C = 2

NUM_CORES = 2
NUM_SUBCORES = 16
NW = NUM_CORES * NUM_SUBCORES
LANES = 16

EDGE_TILE = 128
NTILES = E // EDGE_TILE
BASE_TILES = NTILES // NW
EXTRA = NTILES - BASE_TILES * NW
MAX_EDGES = (BASE_TILES + 1) * EDGE_TILE
BASE_EDGES = BASE_TILES * EDGE_TILE


def _dense_body(x_ref, w1_ref, b1_ref, w2_ref, b2_ref, wc_ref,
                att_ref, z0_ref, z1_ref, info_ref):
    x = x_ref[...]
    hT = jnp.maximum(
        lax.dot_general(w1_ref[...], x, (((0,), (1,)), ((), ())),
                        preferred_element_type=jnp.float32)
        + b1_ref[...][:, None], 0.0)
    logit = lax.dot_general(w2_ref[...], hT, (((0,), (0,)), ((), ())),
                            preferred_element_type=jnp.float32) + b2_ref[0]
    att = jax.nn.sigmoid(logit)
    xwT = lax.dot_general(wc_ref[...], x, (((0,), (1,)), ((), ())),
                          preferred_element_type=jnp.float32)
    att_ref[...] = att.reshape(N)
    z0_ref[...] = (att * xwT[0:1, :]).reshape(N)
    z1_ref[...] = (att * xwT[1:2, :]).reshape(N)
    r = 0.7
    t = att * jnp.log(att / r + 1e-06) + (1.0 - att) * jnp.log((1.0 - att) / (1.0 - r + 1e-06) + 1e-06)
    info_ref[...] = jnp.reshape(jnp.sum(t) / float(N), (1, 1))


def _dense_head(x, W1, b1, W2, b2, Wc):
    return pl.pallas_call(
        _dense_body,
        out_shape=(
            jax.ShapeDtypeStruct((N,), jnp.float32),
            jax.ShapeDtypeStruct((N,), jnp.float32),
            jax.ShapeDtypeStruct((N,), jnp.float32),
            jax.ShapeDtypeStruct((1, 1), jnp.float32),
        ),
    )(x, W1, b1, W2, b2, Wc)


def _sc_body(ei_hbm, att_hbm, z0_hbm, z1_hbm,
             ea_hbm, p0_hbm, p1_hbm,
             ei_v, att_v, z0_v, z1_v, ea_v, acc0_v, acc1_v, sem):
    wid = lax.axis_index("s") * NUM_CORES + lax.axis_index("c")
    ntiles = BASE_TILES + jnp.where(wid < EXTRA, 1, 0)
    base_tile = BASE_TILES * wid + jnp.minimum(wid, EXTRA)
    base_ed = base_tile * EDGE_TILE
    base_eff = jnp.minimum(base_ed, E - MAX_EDGES)
    off = base_ed - base_eff

    cps = [
        pltpu.async_copy(ei_hbm.at[:, pl.ds(base_eff, MAX_EDGES)], ei_v, sem),
        pltpu.async_copy(att_hbm, att_v, sem),
        pltpu.async_copy(z0_hbm, z0_v, sem),
        pltpu.async_copy(z1_hbm, z1_v, sem),
    ]

    def zero_body(i, carry):
        zv = jnp.zeros((LANES,), jnp.float32)
        for k in range(5):
            acc0_v[pl.ds((i * 5 + k) * LANES, LANES)] = zv
            acc1_v[pl.ds((i * 5 + k) * LANES, LANES)] = zv
        return carry

    lax.fori_loop(0, N // (LANES * 5), zero_body, 0)
    for cp in cps:
        cp.wait()

    def do_group(start):
        s = ei_v[0, pl.ds(start, LANES)]
        t = ei_v[1, pl.ds(start, LANES)]
        a_s = plsc.load_gather(att_v, [s])
        a_t = plsc.load_gather(att_v, [t])
        ea_v[pl.ds(start, LANES)] = a_s * a_t
        zs0 = plsc.load_gather(z0_v, [s])
        zs1 = plsc.load_gather(z1_v, [s])
        plsc.addupdate_scatter(acc0_v, [t], zs0)
        plsc.addupdate_scatter(acc1_v, [t], zs1)

    @functools.partial(plsc.parallel_loop, 0, BASE_TILES * (EDGE_TILE // LANES),
                       unroll=4)
    def _(g):
        do_group(off + g * LANES)

    @pl.when(ntiles == BASE_TILES + 1)
    def _():
        def extra_body(k, carry):
            do_group(off + BASE_EDGES + k * LANES)
            return carry
        lax.fori_loop(0, EDGE_TILE // LANES, extra_body, 0)

    pltpu.sync_copy(ea_v.at[pl.ds(off, BASE_EDGES)],
                    ea_hbm.at[pl.ds(base_ed, BASE_EDGES)])

    @pl.when(ntiles == BASE_TILES + 1)
    def _():
        pltpu.sync_copy(ea_v.at[pl.ds(off + BASE_EDGES, EDGE_TILE)],
                        ea_hbm.at[pl.ds(base_ed + BASE_EDGES, EDGE_TILE)])

    pltpu.sync_copy(acc0_v, p0_hbm.at[wid])
    pltpu.sync_copy(acc1_v, p1_hbm.at[wid])


_sc_edges = functools.partial(
    pl.kernel,
    out_type=(
        jax.ShapeDtypeStruct((E,), jnp.float32),
        jax.ShapeDtypeStruct((NW, N), jnp.float32),
        jax.ShapeDtypeStruct((NW, N), jnp.float32),
    ),
    mesh=plsc.VectorSubcoreMesh(core_axis_name="c", subcore_axis_name="s"),
    compiler_params=pltpu.CompilerParams(needs_layout_passes=False),
    scratch_types=[
        pltpu.VMEM((2, MAX_EDGES), jnp.int32),
        pltpu.VMEM((N,), jnp.float32),
        pltpu.VMEM((N,), jnp.float32),
        pltpu.VMEM((N,), jnp.float32),
        pltpu.VMEM((MAX_EDGES,), jnp.float32),
        pltpu.VMEM((N,), jnp.float32),
        pltpu.VMEM((N,), jnp.float32),
        pltpu.SemaphoreType.DMA,
    ],
)(_sc_body)


def _combine_body(p0_ref, p1_ref, att_ref, bc_ref, clf_ref):
    att = att_ref[...][None, :]
    K8 = 2 * NW + 8
    pa = jnp.concatenate(
        [p0_ref[...] * att, p1_ref[...] * att,
         jnp.ones((1, N), jnp.float32),
         jnp.zeros((7, N), jnp.float32)], axis=0)
    rows = lax.broadcasted_iota(jnp.int32, (K8, C), 0)
    cols = lax.broadcasted_iota(jnp.int32, (K8, C), 1)
    bc_row = jnp.where(cols == 0, bc_ref[0], bc_ref[1])
    sel = jnp.where(rows == 2 * NW, bc_row,
                    jnp.where(rows > 2 * NW, 0.0,
                              jnp.where((rows < NW) == (cols == 0), 1.0, 0.0)))
    clf_ref[...] = lax.dot_general(pa, sel, (((0,), (0,)), ((), ())),
                                   preferred_element_type=jnp.float32)


def _combine(p0, p1, att1, bc):
    return pl.pallas_call(
        _combine_body,
        in_specs=[
            pl.BlockSpec(memory_space=pltpu.VMEM),
            pl.BlockSpec(memory_space=pltpu.VMEM),
            pl.BlockSpec(memory_space=pltpu.VMEM),
            pl.BlockSpec(memory_space=pltpu.SMEM),
        ],
        out_shape=jax.ShapeDtypeStruct((N, C), jnp.float32),
    )(p0, p1, att1, bc)


def kernel(x, edge_index, W1, b1, W2, b2, Wc, bc):
    att1, z0, z1, info = _dense_head(x, W1, b1, W2, b2, Wc)
    edge_att, p0, p1 = _sc_edges(edge_index, att1, z0, z1)
    clf_logits = _combine(p0, p1, att1, bc)
    return clf_logits, edge_att, info[0, 0]
